# baseline (speedup 1.0000x reference)
import dataclasses
import functools

import jax
import jax.numpy as jnp
from jax import lax
from jax.experimental import pallas as pl
from jax.experimental.pallas import tpu as pltpu
from jax.experimental.pallas import tpu_sc as plsc

N_BUF = 32768
CHUNK = 2048
NUM_WORKERS = 32
SLICE = N_BUF // NUM_WORKERS
LANES = 16


def _mesh():
    return plsc.VectorSubcoreMesh(core_axis_name="c", subcore_axis_name="s")


def _compiler_params():
    cp = pltpu.CompilerParams()
    if "needs_layout_passes" in pltpu.CompilerParams.__dataclass_fields__:
        cp = dataclasses.replace(cp, needs_layout_passes=False)
    return cp


def _sched_body(nt_hbm, ns_hbm, gt_hbm, gs_hbm, qt_hbm, qs_hbm, scal_hbm,
                ogt_hbm, ogs_hbm, oqt_hbm, oqs_hbm,
                scal_v, src_t_v, src_s_v, b0, b1, b2, b3,
                sem_scal, sem_src, sem_in, sem_out):
    cid = lax.axis_index("c")
    sid = lax.axis_index("s")
    wid = sid * 2 + cid
    base = pl.multiple_of(wid * SLICE, SLICE)
    sl = pl.ds(base, SLICE)
    bufs = (b0, b1, b2, b3)

    dests = (gt_hbm, gs_hbm, qt_hbm, qs_hbm)
    outs = (ogt_hbm, ogs_hbm, oqt_hbm, oqs_hbm)

    scal_load = pltpu.make_async_copy(scal_hbm, scal_v, sem_scal)
    loads = [pltpu.make_async_copy(dests[a].at[sl], bufs[a], sem_in.at[a])
             for a in range(4)]
    src_loads = [pltpu.make_async_copy(nt_hbm, src_t_v, sem_src.at[0]),
                 pltpu.make_async_copy(ns_hbm, src_s_v, sem_src.at[1])]
    stores = [pltpu.make_async_copy(bufs[a], outs[a].at[sl], sem_out.at[a])
              for a in range(4)]

    scal_load.start()
    for c in loads:
        c.start()

    scal_load.wait()
    svec = scal_v[...]
    lane = lax.iota(jnp.int32, LANES)
    nmin = jnp.full((LANES,), jnp.int32(-(2**31)), jnp.int32)

    def _extract(k):
        return jnp.max(jnp.where(lane == k, svec, nmin))

    num = _extract(0)
    start_g = _extract(1)
    start_q = _extract(2)

    starts = (start_g, start_g, start_q, start_q)
    srcs = (src_t_v, src_s_v, src_t_v, src_s_v)
    ov = [jnp.logical_and(base + SLICE > starts[a], base < starts[a] + num)
          for a in range(4)]
    need_t = jnp.logical_or(ov[0], ov[2])
    need_s = jnp.logical_or(ov[1], ov[3])

    @pl.when(need_t)
    def _():
        src_loads[0].start()

    @pl.when(need_s)
    def _():
        src_loads[1].start()

    @pl.when(need_t)
    def _():
        src_loads[0].wait()

    @pl.when(need_s)
    def _():
        src_loads[1].wait()

    for a in range(4):
        loads[a].wait()

        @pl.when(ov[a])
        def _(a=a):
            d_v = bufs[a]
            src_v = srcs[a]
            sv = jnp.full((LANES,), starts[a], jnp.int32)
            ev = sv + jnp.full((LANES,), num, jnp.int32)

            @pl.loop(0, SLICE, step=LANES)
            def _(c0):
                jv = lane + jnp.full((LANES,), base + c0, jnp.int32)
                valid = jnp.logical_and(jv >= sv, jv < ev)
                sidx = jnp.clip(jv - sv, 0, CHUNK - 1)
                gathered = plsc.load_gather(src_v, [sidx])
                cur = d_v[pl.ds(c0, LANES)]
                d_v[pl.ds(c0, LANES)] = jnp.where(valid, gathered, cur)

        stores[a].start()

    for c in stores:
        c.wait()


def kernel(new_tokens, new_token_seq_ids, num_new_tokens,
           generated_tokens, generated_seq_ids, num_generated_tokens,
           queued_tokens, queued_seq_ids, num_queued_tokens):
    num = jnp.minimum(num_new_tokens.astype(jnp.int32), CHUNK)
    start_g = num_generated_tokens.astype(jnp.int32)
    start_q = num_queued_tokens.astype(jnp.int32)
    scal = jnp.zeros((LANES,), jnp.int32)
    scal = scal.at[0].set(num).at[1].set(start_g).at[2].set(start_q)

    buf = jax.ShapeDtypeStruct((N_BUF,), jnp.int32)
    run = functools.partial(
        pl.kernel,
        out_type=[buf, buf, buf, buf],
        mesh=_mesh(),
        compiler_params=_compiler_params(),
        scratch_types=[
            pltpu.VMEM((LANES,), jnp.int32),
            pltpu.VMEM((CHUNK,), jnp.int32),
            pltpu.VMEM((CHUNK,), jnp.int32),
            pltpu.VMEM((SLICE,), jnp.int32),
            pltpu.VMEM((SLICE,), jnp.int32),
            pltpu.VMEM((SLICE,), jnp.int32),
            pltpu.VMEM((SLICE,), jnp.int32),
            pltpu.SemaphoreType.DMA,
            pltpu.SemaphoreType.DMA((2,)),
            pltpu.SemaphoreType.DMA((4,)),
            pltpu.SemaphoreType.DMA((4,)),
        ],
    )(_sched_body)

    og_tok, og_sid, oq_tok, oq_sid = run(
        new_tokens, new_token_seq_ids,
        generated_tokens, generated_seq_ids,
        queued_tokens, queued_seq_ids, scal)

    return (og_tok, og_sid, num_generated_tokens + num_new_tokens,
            oq_tok, oq_sid, num_queued_tokens + num_new_tokens)
